# Initial kernel scaffold; baseline (speedup 1.0000x reference)
import functools

import jax
import jax.numpy as jnp
from jax import lax
from jax.experimental import pallas as pl
from jax.experimental.pallas import tpu as pltpu
from jax.experimental.pallas import tpu_sc as plsc

_N = 100000
_E = 1600000
_PE = 16
_H = 64

_W = 80
_NWIN = _E // _W
_NWORK = 32
_WPT = _NWIN // _NWORK
_G = 25
_NSUP = _WPT // _G
_STRIPE = _N // 16

_R = 2000


def _sc_segsum(glpe, row2d, col2d, zs, zc, ones):
    mesh = plsc.VectorSubcoreMesh(core_axis_name="c", subcore_axis_name="s")

    @functools.partial(
        pl.kernel,
        out_type=[
            jax.ShapeDtypeStruct((2, _N, _PE), jnp.float32),
            jax.ShapeDtypeStruct((2, _N, 1), jnp.float32),
        ],
        mesh=mesh,
        scratch_types=[
            pltpu.VMEM((_G, _W), jnp.int32),
            pltpu.VMEM((_G, _W), jnp.int32),
            pltpu.VMEM((_G * _W, _PE), jnp.float32),
            pltpu.VMEM((_W, 1), jnp.float32),
            pltpu.VMEM_SHARED((_N, _PE), jnp.float32),
            pltpu.VMEM_SHARED((_N, 1), jnp.float32),
            pltpu.SemaphoreType.DMA,
            pltpu.SemaphoreType.DMA,
            pltpu.SemaphoreType.DMA,
        ],
    )
    def k(glpe_hbm, row_hbm, col_hbm, zs_hbm, zc_hbm, ones_hbm,
          s_out, c_out, row_v, col_v, rows_v, ones_v, s_sh, c_sh,
          gsem, ssem, csem):
        c = lax.axis_index("c")
        s = lax.axis_index("s")
        w = c * 16 + s
        stripe = pl.ds(s * _STRIPE, _STRIPE)
        pltpu.sync_copy(zs_hbm, s_sh.at[stripe])
        pltpu.sync_copy(zc_hbm, c_sh.at[stripe])
        pltpu.sync_copy(ones_hbm, ones_v)
        plsc.subcore_barrier()

        @pl.loop(0, _NSUP)
        def _(sc):
            base = w * _WPT + sc * _G
            pltpu.sync_copy(row_hbm.at[pl.ds(base, _G)], row_v)
            pltpu.sync_copy(col_hbm.at[pl.ds(base, _G)], col_v)

            @pl.loop(0, _G)
            def _(j):
                pltpu.async_copy(glpe_hbm.at[row_v.at[j]],
                                 rows_v.at[pl.ds(j * _W, _W)], gsem)

            pltpu.make_async_copy(glpe_hbm.at[pl.ds(0, _G * _W)],
                                  rows_v, gsem).wait()

            @pl.loop(0, _G)
            def _(j):
                pltpu.async_copy(rows_v.at[pl.ds(j * _W, _W)],
                                 s_sh.at[col_v.at[j]], ssem, add=True)
                pltpu.async_copy(ones_v, c_sh.at[col_v.at[j]], csem, add=True)

            pltpu.make_async_copy(rows_v, s_sh.at[pl.ds(0, _G * _W)],
                                  ssem).wait()
            pltpu.make_async_copy(zc_hbm.at[pl.ds(0, _G * _W)],
                                  c_sh.at[pl.ds(0, _G * _W)], csem).wait()

        plsc.subcore_barrier()
        pltpu.sync_copy(s_sh.at[stripe], s_out.at[c, stripe])
        pltpu.sync_copy(c_sh.at[stripe], c_out.at[c, stripe])

    return k(glpe, row2d, col2d, zs, zc, ones)


def _tc_combine(x_clique, deg2d, tree_lpe, seg, cnt, demb_pad,
                w1, b1, w2, b2, tw, tb, lw, lb):
    def body(x_ref, d_ref, t_ref, s_ref, c_ref, de_ref, w1_ref, b1_ref,
             w2_ref, b2_ref, tw_ref, tb_ref, lw_ref, lb_ref, o_ref):
        t3 = jax.nn.relu(
            jnp.dot(de_ref[...], w1_ref[...],
                    preferred_element_type=jnp.float32) + b1_ref[...])
        t4 = jnp.dot(t3, w2_ref[...], preferred_element_type=jnp.float32)
        iota = lax.broadcasted_iota(jnp.int32, (_R, 128), 1)
        onehot = (d_ref[...] == iota).astype(jnp.float32)
        base = (jnp.dot(x_ref[...], w2_ref[...],
                        preferred_element_type=jnp.float32)
                + jnp.dot(onehot, t4, preferred_element_type=jnp.float32)
                + b2_ref[...])
        t = t_ref[...]
        t = jnp.where(t == t, t, 0.0)
        tree_pe = jnp.dot(t, tw_ref[...],
                          preferred_element_type=jnp.float32) + tb_ref[...]
        ssum = s_ref[0] + s_ref[1]
        cn = c_ref[0] + c_ref[1]
        pe = jnp.dot(ssum, lw_ref[...], preferred_element_type=jnp.float32)
        pe = pe / jnp.maximum(cn, 1.0) + lb_ref[...] * jnp.minimum(cn, 1.0)
        o_ref[...] = base + jnp.concatenate([pe, tree_pe], axis=1)

    full = lambda shape: pl.BlockSpec(shape, lambda i: (0,) * len(shape))
    return pl.pallas_call(
        body,
        grid=(_N // _R,),
        in_specs=[
            pl.BlockSpec((_R, _H), lambda i: (i, 0)),
            pl.BlockSpec((_R, 1), lambda i: (i, 0)),
            pl.BlockSpec((_R, _PE), lambda i: (i, 0)),
            pl.BlockSpec((2, _R, _PE), lambda i: (0, i, 0)),
            pl.BlockSpec((2, _R, 1), lambda i: (0, i, 0)),
            full((128, _H)),
            full((_H, _H)),
            full((1, _H)),
            full((_H, _H)),
            full((1, _H)),
            full((_PE, _H // 2)),
            full((1, _H // 2)),
            full((_PE, _H // 2)),
            full((1, _H // 2)),
        ],
        out_specs=pl.BlockSpec((_R, _H), lambda i: (i, 0)),
        out_shape=jax.ShapeDtypeStruct((_N, _H), jnp.float32),
    )(x_clique, deg2d, tree_lpe, seg, cnt, demb_pad,
      w1, b1, w2, b2, tw, tb, lw, lb)


def kernel(x_clique, tree_degree, tree_lpe, graph_lpe, atom2clique_index,
           deg_emb, deg_lin_w, deg_lin_b, deg_merge_w, deg_merge_b,
           tree_lpe_w, tree_lpe_b, lpe_w, lpe_b):
    row2d = atom2clique_index[0].reshape(_NWIN, _W)
    col2d = atom2clique_index[1].reshape(_NWIN, _W)
    zs = jnp.zeros((_STRIPE, _PE), jnp.float32)
    zc = jnp.zeros((_STRIPE, 1), jnp.float32)
    ones = jnp.ones((_W, 1), jnp.float32)
    seg, cnt = _sc_segsum(graph_lpe, row2d, col2d, zs, zc, ones)

    deg2d = tree_degree.reshape(_N, 1)
    demb_pad = jnp.zeros((128, _H), jnp.float32).at[:100].set(deg_emb)
    return _tc_combine(
        x_clique, deg2d, tree_lpe, seg, cnt, demb_pad,
        deg_lin_w, deg_lin_b.reshape(1, _H),
        deg_merge_w, deg_merge_b.reshape(1, _H),
        tree_lpe_w, tree_lpe_b.reshape(1, _H // 2),
        lpe_w, lpe_b.reshape(1, _H // 2))


# speedup vs baseline: 12.5113x; 12.5113x over previous
import functools

import jax
import jax.numpy as jnp
from jax import lax
from jax.experimental import pallas as pl
from jax.experimental.pallas import tpu as pltpu
from jax.experimental.pallas import tpu_sc as plsc

_N = 100000
_E = 1600000
_PE = 16
_H = 64

_W = 125
_WB = 8
_NBLK = _E // (_W * _WB)
_NWORK = 32
_BPT = _NBLK // _NWORK
_G = 1
_NSUP = _BPT // _G
_CH = _G * _WB * _W
_STRIPE = 6256
_STRIPE_LAST = _N - 15 * _STRIPE

_R = 2000


def _sc_segsum(glpe, row2d, col2d, zs, zc, ones):
    mesh = plsc.VectorSubcoreMesh(core_axis_name="c", subcore_axis_name="s")

    @functools.partial(
        pl.kernel,
        out_type=[
            jax.ShapeDtypeStruct((2, _N, _PE), jnp.float32),
            jax.ShapeDtypeStruct((2, _N), jnp.float32),
        ],
        mesh=mesh,
        compiler_params=pltpu.CompilerParams(use_tc_tiling_on_sc=False),
        scratch_types=[
            pltpu.VMEM((_G, _WB, _W), jnp.int32),
            pltpu.VMEM((_G, _WB, _W), jnp.int32),
            pltpu.VMEM((_CH, _PE), jnp.float32),
            pltpu.VMEM((_W,), jnp.float32),
            pltpu.VMEM_SHARED((_N, _PE), jnp.float32),
            pltpu.VMEM_SHARED((_N,), jnp.float32),
            pltpu.SemaphoreType.DMA,
            pltpu.SemaphoreType.DMA,
            pltpu.SemaphoreType.DMA,
        ],
    )
    def k(glpe_hbm, row_hbm, col_hbm, zs_hbm, zc_hbm, ones_hbm,
          s_out, c_out, row_v, col_v, rows_v, ones_v, s_sh, c_sh,
          gsem, ssem, csem):
        c = lax.axis_index("c")
        s = lax.axis_index("s")
        w = c * 16 + s
        @pl.when(s < 15)
        def _():
            stripe = pl.ds(s * _STRIPE, _STRIPE)
            pltpu.sync_copy(zs_hbm, s_sh.at[stripe])
            pltpu.sync_copy(zc_hbm, c_sh.at[stripe])

        @pl.when(s == 15)
        def _():
            stripe = pl.ds(15 * _STRIPE, _STRIPE_LAST)
            pltpu.sync_copy(zs_hbm.at[pl.ds(0, _STRIPE_LAST)], s_sh.at[stripe])
            pltpu.sync_copy(zc_hbm.at[pl.ds(0, _STRIPE_LAST)], c_sh.at[stripe])

        pltpu.sync_copy(ones_hbm, ones_v)
        plsc.subcore_barrier()

        @pl.loop(0, _NSUP)
        def _(sc):
            base = w * _BPT + sc * _G
            pltpu.sync_copy(row_hbm.at[pl.ds(base, _G)], row_v)
            pltpu.sync_copy(col_hbm.at[pl.ds(base, _G)], col_v)

            @pl.loop(0, _G)
            def _(g):
                @pl.loop(0, _WB)
                def _(j):
                    pltpu.async_copy(
                        glpe_hbm.at[row_v.at[g, j]],
                        rows_v.at[pl.ds((g * _WB + j) * _W, _W)], gsem)

            pltpu.make_async_copy(glpe_hbm.at[pl.ds(0, _CH)],
                                  rows_v, gsem).wait()

            @pl.loop(0, _G)
            def _(g):
                @pl.loop(0, _WB)
                def _(j):
                    src = rows_v.at[pl.ds((g * _WB + j) * _W, _W)]
                    pltpu.async_copy(src, s_sh.at[col_v.at[g, j]],
                                     ssem, add=True)
                    pltpu.async_copy(ones_v, c_sh.at[col_v.at[g, j]],
                                     csem, add=True)

            pltpu.make_async_copy(rows_v, s_sh.at[pl.ds(0, _CH)],
                                  ssem).wait()
            pltpu.make_async_copy(zc_hbm.at[pl.ds(0, _CH)],
                                  c_sh.at[pl.ds(0, _CH)], csem).wait()

        plsc.subcore_barrier()

        @pl.when(s < 15)
        def _():
            stripe = pl.ds(s * _STRIPE, _STRIPE)
            pltpu.sync_copy(s_sh.at[stripe], s_out.at[c, stripe])
            pltpu.sync_copy(c_sh.at[stripe], c_out.at[c, stripe])

        @pl.when(s == 15)
        def _():
            stripe = pl.ds(15 * _STRIPE, _STRIPE_LAST)
            pltpu.sync_copy(s_sh.at[stripe], s_out.at[c, stripe])
            pltpu.sync_copy(c_sh.at[stripe], c_out.at[c, stripe])

    return k(glpe, row2d, col2d, zs, zc, ones)


def _tc_combine(x_clique, deg2d, tree_lpe, seg, cnt, demb_pad,
                w1, b1, w2, b2, tw, tb, lw, lb):
    def body(x_ref, d_ref, t_ref, s_ref, c_ref, de_ref, w1_ref, b1_ref,
             w2_ref, b2_ref, tw_ref, tb_ref, lw_ref, lb_ref, o_ref):
        t3 = jax.nn.relu(
            jnp.dot(de_ref[...], w1_ref[...],
                    preferred_element_type=jnp.float32) + b1_ref[...])
        t4 = jnp.dot(t3, w2_ref[...], preferred_element_type=jnp.float32)
        iota = lax.broadcasted_iota(jnp.int32, (_R, 128), 1)
        onehot = (d_ref[...] == iota).astype(jnp.float32)
        base = (jnp.dot(x_ref[...], w2_ref[...],
                        preferred_element_type=jnp.float32)
                + jnp.dot(onehot, t4, preferred_element_type=jnp.float32)
                + b2_ref[...])
        t = t_ref[...]
        t = jnp.where(t == t, t, 0.0)
        tree_pe = jnp.dot(t, tw_ref[...],
                          preferred_element_type=jnp.float32) + tb_ref[...]
        ssum = s_ref[0] + s_ref[1]
        cn = c_ref[0] + c_ref[1]
        pe = jnp.dot(ssum, lw_ref[...], preferred_element_type=jnp.float32)
        pe = pe / jnp.maximum(cn, 1.0) + lb_ref[...] * jnp.minimum(cn, 1.0)
        o_ref[...] = base + jnp.concatenate([pe, tree_pe], axis=1)

    full = lambda shape: pl.BlockSpec(shape, lambda i: (0,) * len(shape))
    return pl.pallas_call(
        body,
        grid=(_N // _R,),
        in_specs=[
            pl.BlockSpec((_R, _H), lambda i: (i, 0)),
            pl.BlockSpec((_R, 1), lambda i: (i, 0)),
            pl.BlockSpec((_R, _PE), lambda i: (i, 0)),
            pl.BlockSpec((2, _R, _PE), lambda i: (0, i, 0)),
            pl.BlockSpec((2, _R, 1), lambda i: (0, i, 0)),
            full((128, _H)),
            full((_H, _H)),
            full((1, _H)),
            full((_H, _H)),
            full((1, _H)),
            full((_PE, _H // 2)),
            full((1, _H // 2)),
            full((_PE, _H // 2)),
            full((1, _H // 2)),
        ],
        out_specs=pl.BlockSpec((_R, _H), lambda i: (i, 0)),
        out_shape=jax.ShapeDtypeStruct((_N, _H), jnp.float32),
    )(x_clique, deg2d, tree_lpe, seg, cnt, demb_pad,
      w1, b1, w2, b2, tw, tb, lw, lb)


def kernel(x_clique, tree_degree, tree_lpe, graph_lpe, atom2clique_index,
           deg_emb, deg_lin_w, deg_lin_b, deg_merge_w, deg_merge_b,
           tree_lpe_w, tree_lpe_b, lpe_w, lpe_b):
    row3d = atom2clique_index[0].reshape(_NBLK, _WB, _W)
    col3d = atom2clique_index[1].reshape(_NBLK, _WB, _W)
    zs = jnp.zeros((_STRIPE, _PE), jnp.float32)
    zc = jnp.zeros((_STRIPE,), jnp.float32)
    ones = jnp.ones((_W,), jnp.float32)
    seg, cnt = _sc_segsum(graph_lpe, row3d, col3d, zs, zc, ones)
    cnt = cnt.reshape(2, _N, 1)

    deg2d = tree_degree.reshape(_N, 1)
    demb_pad = jnp.zeros((128, _H), jnp.float32).at[:100].set(deg_emb)
    return _tc_combine(
        x_clique, deg2d, tree_lpe, seg, cnt, demb_pad,
        deg_lin_w, deg_lin_b.reshape(1, _H),
        deg_merge_w, deg_merge_b.reshape(1, _H),
        tree_lpe_w, tree_lpe_b.reshape(1, _H // 2),
        lpe_w, lpe_b.reshape(1, _H // 2))
